# baseline (speedup 1.0000x reference)
import functools

import jax
import jax.numpy as jnp
from jax import lax
from jax.experimental import pallas as pl
from jax.experimental.pallas import tpu as pltpu
from jax.experimental.pallas import tpu_sc as plsc

VOCAB = 1000000
EMBED = 64
SEQ = 77
BATCH = 4096

NC = 2
NS = 16
NW = NC * NS

SEQ_PER_W = BATCH // NW
CH_SEQ = 8
CH = CH_SEQ * SEQ
N_CHUNK = SEQ_PER_W // CH_SEQ
ROWS_PER_W = SEQ_PER_W * SEQ
G = 88
N_G = CH // G


def _make_kernel():
    mesh = plsc.VectorSubcoreMesh(core_axis_name="c", subcore_axis_name="s")
    B = BATCH * SEQ

    @functools.partial(
        pl.kernel,
        mesh=mesh,
        out_type=jax.ShapeDtypeStruct((B, EMBED), jnp.float32),
        scratch_types=[
            pltpu.VMEM((CH,), jnp.int32),
            pltpu.VMEM((CH, EMBED), jnp.float32),
            pltpu.VMEM((CH, EMBED), jnp.float32),
            pltpu.SemaphoreType.DMA,
        ],
        compiler_params=pltpu.CompilerParams(use_tc_tiling_on_sc=False),
    )
    def body(ids_hbm, table_hbm, pos_hbm, out_hbm, idx_v, buf, pos_v, sem):
        wid = lax.axis_index("s") * NC + lax.axis_index("c")
        base = wid * ROWS_PER_W

        pltpu.sync_copy(pos_hbm, pos_v)

        def chunk_body(c, carry):
            off = base + c * CH
            pltpu.sync_copy(ids_hbm.at[pl.ds(off, CH)], idx_v)
            copies = []
            for j in range(N_G):
                copies.append(pltpu.async_copy(
                    table_hbm.at[idx_v.at[pl.ds(j * G, G)]],
                    buf.at[pl.ds(j * G, G)],
                    sem,
                ))
            for cp in copies:
                cp.wait()

            def add_body(r, carry2):
                for q in range(EMBED // 16):
                    sl = pl.ds(q * 16, 16)
                    buf[r, sl] += pos_v[r, sl]
                return carry2

            lax.fori_loop(0, CH, add_body, 0)
            pltpu.sync_copy(buf, out_hbm.at[pl.ds(off, CH)])
            return carry

        lax.fori_loop(0, N_CHUNK, chunk_body, 0)

    return body


_sc_kernel = _make_kernel()


def kernel(input_ids, token_embedding, position_embedding):
    ids_flat = input_ids.reshape(-1).astype(jnp.int32)
    pos_tile = jnp.tile(position_embedding[:SEQ], (CH_SEQ, 1))
    out = _sc_kernel(ids_flat, token_embedding, pos_tile)
    return out.reshape(BATCH, SEQ, EMBED)
